# baseline (speedup 1.0000x reference)
import functools

import jax
import jax.numpy as jnp
from jax import lax
from jax.experimental import pallas as pl
from jax.experimental.pallas import tpu as pltpu
from jax.experimental.pallas import tpu_sc as plsc

N_USERS = 25000
N_ITEMS = 25000
N = 50000
D = 32
NI = 128
L = 2
E = 800000

NC = 2
NS = 16
LANES = 16
NW = NC * NS
CH = 256
NK = 102
NCHP = NK * NW
EPAD = NCHP * CH
NPAD = 51200
STRIPE = NPAD // NS
NBUF = 2

f32 = jnp.float32
i32 = jnp.int32

_MESH = plsc.VectorSubcoreMesh(core_axis_name="c", subcore_axis_name="s")
_SC_PARAMS = pltpu.CompilerParams(use_tc_tiling_on_sc=False,
                                  needs_layout_passes=False)

_SEMS = [pltpu.SemaphoreType.DMA] * (2 * NBUF)


def _ids():
    c = lax.axis_index("c")
    s = lax.axis_index("s")
    wid = s * NC + c
    return c, s, wid


def _ring(nk, fetch_idx, issue_gather, wait_gather, consume):
    fetch_idx(0, 0)
    issue_gather(0, 0)

    @pl.loop(0, nk // 2)
    def _(g):
        for b in range(2):
            k = g * 2 + b

            @pl.when(k + 1 < nk)
            def _():
                fetch_idx(k + 1, 1 - b)
                issue_gather(k + 1, 1 - b)

            wait_gather(k, b)
            consume(k, b)


@functools.partial(
    pl.kernel,
    out_type=jax.ShapeDtypeStruct((NC, NPAD), f32),
    mesh=_MESH,
    compiler_params=_SC_PARAMS,
    scratch_types=[pltpu.VMEM((NBUF, 2, CH), i32), pltpu.VMEM((CH,), f32),
                   pltpu.VMEM_SHARED((NPAD,), f32)] + _SEMS,
)
def _sc_deg(ht_hbm, z1_hbm, out_hbm, ib, ones_v, deg_sh, *sems):
    c, s, wid = _ids()
    st = pl.ds(s * STRIPE, STRIPE)
    pltpu.sync_copy(z1_hbm.at[st], deg_sh.at[st])
    for i in range(CH // LANES):
        ones_v[pl.ds(i * LANES, LANES)] = jnp.ones((LANES,), f32)
    plsc.subcore_barrier()
    sgm, ssm = sems[:NBUF], sems[NBUF:]
    base = wid * NK

    def fi(k, b):
        pltpu.sync_copy(ht_hbm.at[base + k], ib.at[b])

    def ig(k, b):
        pass

    def wg(k, b):
        pass

    def cs(k, b):
        pltpu.sync_copy(ones_v, deg_sh.at[ib.at[b, 0]], add=True)

    _ring(NK, fi, ig, wg, cs)
    plsc.subcore_barrier()
    pltpu.sync_copy(deg_sh.at[st], out_hbm.at[c, st])


@functools.partial(
    pl.kernel,
    out_type=jax.ShapeDtypeStruct((NC, NPAD, D), f32),
    mesh=_MESH,
    compiler_params=_SC_PARAMS,
    scratch_types=[pltpu.VMEM((NBUF, 2, CH), i32),
                   pltpu.VMEM((NBUF, CH, D), f32),
                   pltpu.VMEM_SHARED((NPAD, D), f32)] + _SEMS,
)
def _sc_spmm(ht_hbm, tab_hbm, z2_hbm, out_hbm, ib, rows, acc_sh, *sems):
    c, s, wid = _ids()
    st = pl.ds(s * STRIPE, STRIPE)
    pltpu.sync_copy(z2_hbm.at[st], acc_sh.at[st])
    plsc.subcore_barrier()
    sgm, ssm = sems[:NBUF], sems[NBUF:]
    base = wid * NK

    def fi(k, b):
        pltpu.sync_copy(ht_hbm.at[base + k], ib.at[b])

    def ig(k, b):
        pltpu.async_copy(tab_hbm.at[ib.at[b, 1]], rows.at[b], sgm[b])

    def wg(k, b):
        pltpu.make_async_copy(tab_hbm.at[ib.at[b, 1]], rows.at[b],
                              sgm[b]).wait()

    def cs(k, b):
        pltpu.sync_copy(rows.at[b], acc_sh.at[ib.at[b, 0]], add=True)

    _ring(NK, fi, ig, wg, cs)
    plsc.subcore_barrier()
    pltpu.sync_copy(acc_sh.at[st], out_hbm.at[c, st])


@functools.partial(
    pl.kernel,
    out_type=(
        jax.ShapeDtypeStruct((NCHP, CH), f32),
        jax.ShapeDtypeStruct((NCHP, CH), f32),
        jax.ShapeDtypeStruct((NC, NPAD), f32),
        jax.ShapeDtypeStruct((NC, NPAD), f32),
    ),
    mesh=_MESH,
    compiler_params=_SC_PARAMS,
    scratch_types=[
        pltpu.VMEM((NBUF, 2, CH), i32),
        pltpu.VMEM((NBUF, CH, 2 * D), f32),
        pltpu.VMEM((NBUF, CH, 2 * D), f32),
        pltpu.VMEM((NBUF, CH), f32),
        pltpu.VMEM((NBUF, CH), f32),
        pltpu.VMEM_SHARED((NPAD,), f32),
        pltpu.VMEM_SHARED((NPAD,), f32),
    ] + _SEMS,
)
def _sc_alpha(ht_hbm, c2_hbm, z1_hbm, ag_hbm, ai_hbm, dg_hbm, di_hbm,
              ib, rh, rt, ag, ai, dg_sh, di_sh, *sems):
    c, s, wid = _ids()
    st = pl.ds(s * STRIPE, STRIPE)
    pltpu.sync_copy(z1_hbm.at[st], dg_sh.at[st])
    pltpu.sync_copy(z1_hbm.at[st], di_sh.at[st])
    plsc.subcore_barrier()
    sgm, ssm = sems[:NBUF], sems[NBUF:]
    lane = lax.iota(i32, LANES)
    base = wid * NK

    def fi(k, b):
        pltpu.sync_copy(ht_hbm.at[base + k], ib.at[b])

    def ig(k, b):
        pltpu.async_copy(c2_hbm.at[ib.at[b, 0]], rh.at[b], sgm[b])
        pltpu.async_copy(c2_hbm.at[ib.at[b, 1]], rt.at[b], sgm[b])

    def wg(k, b):
        pltpu.make_async_copy(c2_hbm.at[ib.at[b, 0]], rh.at[b],
                              sgm[b]).wait()
        pltpu.make_async_copy(c2_hbm.at[ib.at[b, 1]], rt.at[b],
                              sgm[b]).wait()

    def cs(k, b):
        @pl.when(k >= 2)
        def _():
            kp = k - 2
            pltpu.make_async_copy(ag.at[b], ag_hbm.at[base + kp],
                                  ssm[b]).wait()
            pltpu.make_async_copy(ai.at[b], ai_hbm.at[base + kp],
                                  ssm[b]).wait()

        @pl.loop(0, CH // LANES)
        def _(g):
            rowv = g * LANES + lane
            accg = jnp.zeros((LANES,), f32)
            acci = jnp.zeros((LANES,), f32)
            for d in range(D):
                cold = jnp.full((LANES,), d, i32)
                accg = accg + (plsc.load_gather(rh.at[b], [rowv, cold])
                               * plsc.load_gather(rt.at[b], [rowv, cold]))
            for d in range(D, 2 * D):
                cold = jnp.full((LANES,), d, i32)
                acci = acci + (plsc.load_gather(rh.at[b], [rowv, cold])
                               * plsc.load_gather(rt.at[b], [rowv, cold]))
            ag[b, pl.ds(g * LANES, LANES)] = (accg + 1.0) * 0.5
            ai[b, pl.ds(g * LANES, LANES)] = (acci + 1.0) * 0.5

        pltpu.async_copy(ag.at[b], ag_hbm.at[base + k], ssm[b])
        pltpu.async_copy(ai.at[b], ai_hbm.at[base + k], ssm[b])
        pltpu.sync_copy(ag.at[b], dg_sh.at[ib.at[b, 0]], add=True)
        pltpu.sync_copy(ai.at[b], di_sh.at[ib.at[b, 0]], add=True)

    _ring(NK, fi, ig, wg, cs)
    for b in range(2):
        kp = NK - 2 + b
        pltpu.make_async_copy(ag.at[b], ag_hbm.at[base + kp], ssm[b]).wait()
        pltpu.make_async_copy(ai.at[b], ai_hbm.at[base + kp], ssm[b]).wait()
    plsc.subcore_barrier()
    pltpu.sync_copy(dg_sh.at[st], dg_hbm.at[c, st])
    pltpu.sync_copy(di_sh.at[st], di_hbm.at[c, st])


@functools.partial(
    pl.kernel,
    out_type=jax.ShapeDtypeStruct((NC, NPAD, D), f32),
    mesh=_MESH,
    compiler_params=_SC_PARAMS,
    scratch_types=[
        pltpu.VMEM((NBUF, 2, CH), i32),
        pltpu.VMEM((NBUF, CH, D), f32),
        pltpu.VMEM((NBUF, CH), f32),
        pltpu.VMEM((NBUF, CH), f32),
        pltpu.VMEM((NBUF, CH), f32),
        pltpu.VMEM((NBUF, CH), f32),
        pltpu.VMEM_SHARED((NPAD, D), f32),
    ] + _SEMS,
)
def _sc_gaia(ht_hbm, emb_hbm, dig_hbm, dii_hbm, ag_hbm, ai_hbm, z2_hbm,
             out_hbm, ib, rows, agb, aib, dgb, dib, acc_sh, *sems):
    c, s, wid = _ids()
    st = pl.ds(s * STRIPE, STRIPE)
    pltpu.sync_copy(z2_hbm.at[st], acc_sh.at[st])
    plsc.subcore_barrier()
    sgm, ssm = sems[:NBUF], sems[NBUF:]
    base = wid * NK

    def fi(k, b):
        pltpu.sync_copy(ht_hbm.at[base + k], ib.at[b])

    def ig(k, b):
        pltpu.async_copy(emb_hbm.at[ib.at[b, 1]], rows.at[b], sgm[b])
        pltpu.async_copy(dig_hbm.at[ib.at[b, 0]], dgb.at[b], sgm[b])
        pltpu.async_copy(dii_hbm.at[ib.at[b, 0]], dib.at[b], sgm[b])
        pltpu.async_copy(ag_hbm.at[base + k], agb.at[b], sgm[b])
        pltpu.async_copy(ai_hbm.at[base + k], aib.at[b], sgm[b])

    def wg(k, b):
        pltpu.make_async_copy(emb_hbm.at[ib.at[b, 1]], rows.at[b],
                              sgm[b]).wait()
        pltpu.make_async_copy(dig_hbm.at[ib.at[b, 0]], dgb.at[b],
                              sgm[b]).wait()
        pltpu.make_async_copy(dii_hbm.at[ib.at[b, 0]], dib.at[b],
                              sgm[b]).wait()
        pltpu.make_async_copy(ag_hbm.at[base + k], agb.at[b], sgm[b]).wait()
        pltpu.make_async_copy(ai_hbm.at[base + k], aib.at[b], sgm[b]).wait()

    def cs(k, b):
        @pl.loop(0, CH // LANES)
        def _(g):
            sl = pl.ds(g * LANES, LANES)
            vvec = (dgb[b, sl] * agb[b, sl] + dib[b, sl] * aib[b, sl])
            for j in range(LANES):
                e = g * LANES + j
                v = vvec[j]
                rows[b, e, pl.ds(0, 16)] = rows[b, e, pl.ds(0, 16)] * v
                rows[b, e, pl.ds(16, 16)] = rows[b, e, pl.ds(16, 16)] * v

        pltpu.sync_copy(rows.at[b], acc_sh.at[ib.at[b, 0]], add=True)

    _ring(NK, fi, ig, wg, cs)
    plsc.subcore_barrier()
    pltpu.sync_copy(acc_sh.at[st], out_hbm.at[c, st])


_B1 = 400
_B2 = 1000


def _tc_pre_body(degp_ref, emb_ref, dis_ref, embs_ref):
    dp = degp_ref[...]
    deg = dp[0] + dp[1]
    safe = jnp.where(deg > 0, deg, 1.0)
    dis = jnp.where(deg > 0, lax.rsqrt(safe), 0.0)
    dis_ref[...] = dis
    embs_ref[...] = emb_ref[...] * dis


def _tc_pre(deg_parts, emb):
    return pl.pallas_call(
        _tc_pre_body,
        grid=(N // _B1,),
        in_specs=[
            pl.BlockSpec((NC, _B1, 1), lambda i: (0, i, 0)),
            pl.BlockSpec((_B1, D), lambda i: (i, 0)),
        ],
        out_specs=[
            pl.BlockSpec((_B1, 1), lambda i: (i, 0)),
            pl.BlockSpec((_B1, D), lambda i: (i, 0)),
        ],
        out_shape=[
            jax.ShapeDtypeStruct((N, 1), f32),
            jax.ShapeDtypeStruct((N, D), f32),
        ],
    )(deg_parts.reshape(NC, NPAD, 1), emb)


def _tc_int_body(emb_ref, wu_ref, wi_ref, int_ref, intn_ref):
    pid = pl.program_id(0)
    w = jnp.where(pid < N_USERS // _B2, wu_ref[...], wi_ref[...])
    x = emb_ref[...]
    logits = jnp.dot(x, w, preferred_element_type=f32)
    m = jnp.max(logits, axis=1, keepdims=True)
    p = jnp.exp(logits - m)
    sm = p / jnp.sum(p, axis=1, keepdims=True)
    it = lax.dot_general(sm, w, (((1,), (1,)), ((), ())),
                         preferred_element_type=f32)
    int_ref[...] = it
    n = jnp.sqrt(jnp.sum(it * it, axis=1, keepdims=True))
    intn_ref[...] = it / jnp.maximum(n, 1e-12)


def _tc_int(emb, wu, wi):
    return pl.pallas_call(
        _tc_int_body,
        grid=(N // _B2,),
        in_specs=[
            pl.BlockSpec((_B2, D), lambda i: (i, 0)),
            pl.BlockSpec((D, NI), lambda i: (0, 0)),
            pl.BlockSpec((D, NI), lambda i: (0, 0)),
        ],
        out_specs=[
            pl.BlockSpec((_B2, D), lambda i: (i, 0)),
            pl.BlockSpec((_B2, D), lambda i: (i, 0)),
        ],
        out_shape=[
            jax.ShapeDtypeStruct((N, D), f32),
            jax.ShapeDtypeStruct((N, D), f32),
        ],
    )(emb, wu, wi)


def _tc_mid_body(sg_ref, dis_ref, intn_ref, gnn_ref, c2_ref):
    sg = sg_ref[...]
    gnn = (sg[0] + sg[1]) * dis_ref[...]
    gnn_ref[...] = gnn
    n = jnp.sqrt(jnp.sum(gnn * gnn, axis=1, keepdims=True))
    gnnn = gnn / jnp.maximum(n, 1e-12)
    c2_ref[...] = jnp.concatenate([gnnn, intn_ref[...]], axis=1)


def _tc_mid(sg_parts, dis, intn):
    return pl.pallas_call(
        _tc_mid_body,
        grid=(N // _B1,),
        in_specs=[
            pl.BlockSpec((NC, _B1, D), lambda i: (0, i, 0)),
            pl.BlockSpec((_B1, 1), lambda i: (i, 0)),
            pl.BlockSpec((_B1, D), lambda i: (i, 0)),
        ],
        out_specs=[
            pl.BlockSpec((_B1, D), lambda i: (i, 0)),
            pl.BlockSpec((_B1, 2 * D), lambda i: (i, 0)),
        ],
        out_shape=[
            jax.ShapeDtypeStruct((N, D), f32),
            jax.ShapeDtypeStruct((N, 2 * D), f32),
        ],
    )(sg_parts, dis, intn)


def _tc_dinv_body(dgp_ref, dip_ref, dig_ref, dii_ref):
    dg = dgp_ref[...]
    di = dip_ref[...]
    g = dg[0] + dg[1]
    i = di[0] + di[1]
    dig_ref[...] = jnp.where(g != 0, 1.0 / jnp.where(g != 0, g, 1.0), 0.0)
    dii_ref[...] = jnp.where(i != 0, 1.0 / jnp.where(i != 0, i, 1.0), 0.0)


def _tc_dinv(dg_parts, di_parts):
    return pl.pallas_call(
        _tc_dinv_body,
        grid=(N // _B1,),
        in_specs=[
            pl.BlockSpec((NC, _B1, 1), lambda i: (0, i, 0)),
            pl.BlockSpec((NC, _B1, 1), lambda i: (0, i, 0)),
        ],
        out_specs=[
            pl.BlockSpec((_B1, 1), lambda i: (i, 0)),
            pl.BlockSpec((_B1, 1), lambda i: (i, 0)),
        ],
        out_shape=[
            jax.ShapeDtypeStruct((N, 1), f32),
            jax.ShapeDtypeStruct((N, 1), f32),
        ],
    )(dg_parts.reshape(NC, NPAD, 1), di_parts.reshape(NC, NPAD, 1))


def _tc_comb_body(gnn_ref, int_ref, ga_ref, emb_ref, acc_ref, dis_ref,
                  embn_ref, accn_ref, embsn_ref):
    ga = ga_ref[...]
    e2 = gnn_ref[...] + int_ref[...] + ga[0] + ga[1] + emb_ref[...]
    embn_ref[...] = e2
    accn_ref[...] = acc_ref[...] + e2
    embsn_ref[...] = e2 * dis_ref[...]


def _tc_comb(gnn, int_emb, ga_parts, emb, acc, dis):
    return pl.pallas_call(
        _tc_comb_body,
        grid=(N // _B1,),
        in_specs=[
            pl.BlockSpec((_B1, D), lambda i: (i, 0)),
            pl.BlockSpec((_B1, D), lambda i: (i, 0)),
            pl.BlockSpec((NC, _B1, D), lambda i: (0, i, 0)),
            pl.BlockSpec((_B1, D), lambda i: (i, 0)),
            pl.BlockSpec((_B1, D), lambda i: (i, 0)),
            pl.BlockSpec((_B1, 1), lambda i: (i, 0)),
        ],
        out_specs=[
            pl.BlockSpec((_B1, D), lambda i: (i, 0)),
            pl.BlockSpec((_B1, D), lambda i: (i, 0)),
            pl.BlockSpec((_B1, D), lambda i: (i, 0)),
        ],
        out_shape=[
            jax.ShapeDtypeStruct((N, D), f32),
            jax.ShapeDtypeStruct((N, D), f32),
            jax.ShapeDtypeStruct((N, D), f32),
        ],
    )(gnn, int_emb, ga_parts, emb, acc, dis)


def kernel(user_emb, item_emb, user_intent, item_intent, all_h_list,
           all_t_list):
    emb = jnp.concatenate([user_emb, item_emb], axis=0)
    h2 = jnp.pad(all_h_list, (0, EPAD - E),
                 constant_values=N).reshape(NCHP, CH)
    t2 = jnp.pad(all_t_list, (0, EPAD - E),
                 constant_values=0).reshape(NCHP, CH)
    ht2 = jnp.stack([h2, t2], axis=1)
    z1 = jnp.zeros((NPAD,), f32)
    z2 = jnp.zeros((NPAD, D), f32)

    deg_parts = _sc_deg(ht2, z1)
    dis, embs = _tc_pre(deg_parts, emb)

    acc = emb
    for _ in range(L):
        int_emb, intn = _tc_int(emb, user_intent, item_intent)
        sg_parts = _sc_spmm(ht2, embs, z2)
        gnn, c2 = _tc_mid(sg_parts, dis, intn)
        ag, ai, dg_parts, di_parts = _sc_alpha(ht2, c2, z1)
        dig, dii = _tc_dinv(dg_parts, di_parts)
        dig_p = jnp.pad(dig.reshape(N), (0, NPAD - N))
        dii_p = jnp.pad(dii.reshape(N), (0, NPAD - N))
        ga_parts = _sc_gaia(ht2, emb, dig_p, dii_p, ag, ai, z2)
        emb, acc, embs = _tc_comb(gnn, int_emb, ga_parts, emb, acc, dis)
    return acc


# speedup vs baseline: 1.3666x; 1.3666x over previous
import functools

import jax
import jax.numpy as jnp
from jax import lax
from jax.experimental import pallas as pl
from jax.experimental.pallas import tpu as pltpu
from jax.experimental.pallas import tpu_sc as plsc

N_USERS = 25000
N_ITEMS = 25000
N = 50000
D = 32
NI = 128
L = 2
E = 800000

NC = 2
NS = 16
LANES = 16
NW = NC * NS
CH = 256
NK = 102
NCHP = NK * NW
EPAD = NCHP * CH
NPAD = 51200
STRIPE = NPAD // NS
NBUF = 2

f32 = jnp.float32
i32 = jnp.int32

_MESH = plsc.VectorSubcoreMesh(core_axis_name="c", subcore_axis_name="s")
_SC_PARAMS = pltpu.CompilerParams(use_tc_tiling_on_sc=False,
                                  needs_layout_passes=False)

_SEMS = [pltpu.SemaphoreType.DMA] * (2 * NBUF)


def _ids():
    c = lax.axis_index("c")
    s = lax.axis_index("s")
    wid = s * NC + c
    return c, s, wid


def _ring(nk, fetch_idx, issue_gather, wait_gather, consume):
    fetch_idx(0, 0)
    issue_gather(0, 0)

    @pl.loop(0, nk // 2)
    def _(g):
        for b in range(2):
            k = g * 2 + b

            @pl.when(k + 1 < nk)
            def _():
                fetch_idx(k + 1, 1 - b)
                issue_gather(k + 1, 1 - b)

            wait_gather(k, b)
            consume(k, b)


@functools.partial(
    pl.kernel,
    out_type=jax.ShapeDtypeStruct((NC, NPAD), f32),
    mesh=_MESH,
    compiler_params=_SC_PARAMS,
    scratch_types=[pltpu.VMEM((NBUF, 2, CH), i32), pltpu.VMEM((CH,), f32),
                   pltpu.VMEM_SHARED((NPAD,), f32)] + _SEMS,
)
def _sc_deg(ht_hbm, z1_hbm, out_hbm, ib, ones_v, deg_sh, *sems):
    c, s, wid = _ids()
    st = pl.ds(s * STRIPE, STRIPE)
    pltpu.sync_copy(z1_hbm.at[st], deg_sh.at[st])
    for i in range(CH // LANES):
        ones_v[pl.ds(i * LANES, LANES)] = jnp.ones((LANES,), f32)
    plsc.subcore_barrier()
    sgm, ssm = sems[:NBUF], sems[NBUF:]
    base = wid * NK

    def fi(k, b):
        pltpu.sync_copy(ht_hbm.at[base + k], ib.at[b])

    def ig(k, b):
        pass

    def wg(k, b):
        pass

    def cs(k, b):
        pltpu.sync_copy(ones_v, deg_sh.at[ib.at[b, 0]], add=True)

    _ring(NK, fi, ig, wg, cs)
    plsc.subcore_barrier()
    pltpu.sync_copy(deg_sh.at[st], out_hbm.at[c, st])


@functools.partial(
    pl.kernel,
    out_type=jax.ShapeDtypeStruct((NC, NPAD, D), f32),
    mesh=_MESH,
    compiler_params=_SC_PARAMS,
    scratch_types=[pltpu.VMEM((NBUF, 2, CH), i32),
                   pltpu.VMEM((NBUF, CH, D), f32),
                   pltpu.VMEM_SHARED((NPAD, D), f32)] + _SEMS,
)
def _sc_spmm(ht_hbm, tab_hbm, z2_hbm, out_hbm, ib, rows, acc_sh, *sems):
    c, s, wid = _ids()
    st = pl.ds(s * STRIPE, STRIPE)
    pltpu.sync_copy(z2_hbm.at[st], acc_sh.at[st])
    plsc.subcore_barrier()
    sgm, ssm = sems[:NBUF], sems[NBUF:]
    base = wid * NK

    def fi(k, b):
        pltpu.sync_copy(ht_hbm.at[base + k], ib.at[b])

    def ig(k, b):
        pltpu.async_copy(tab_hbm.at[ib.at[b, 1]], rows.at[b], sgm[b])

    def wg(k, b):
        pltpu.make_async_copy(tab_hbm.at[ib.at[b, 1]], rows.at[b],
                              sgm[b]).wait()

    def cs(k, b):
        pltpu.sync_copy(rows.at[b], acc_sh.at[ib.at[b, 0]], add=True)

    _ring(NK, fi, ig, wg, cs)
    plsc.subcore_barrier()
    pltpu.sync_copy(acc_sh.at[st], out_hbm.at[c, st])


@functools.partial(
    pl.kernel,
    out_type=(
        jax.ShapeDtypeStruct((NCHP, CH), f32),
        jax.ShapeDtypeStruct((NCHP, CH), f32),
        jax.ShapeDtypeStruct((NC, NPAD), f32),
        jax.ShapeDtypeStruct((NC, NPAD), f32),
    ),
    mesh=_MESH,
    compiler_params=_SC_PARAMS,
    scratch_types=[
        pltpu.VMEM((NBUF, 2, CH), i32),
        pltpu.VMEM((NBUF, CH, 2 * D), f32),
        pltpu.VMEM((NBUF, CH, 2 * D), f32),
        pltpu.VMEM((NBUF, CH), f32),
        pltpu.VMEM((NBUF, CH), f32),
        pltpu.VMEM_SHARED((NPAD,), f32),
        pltpu.VMEM_SHARED((NPAD,), f32),
    ] + _SEMS,
)
def _sc_alpha(ht_hbm, c2_hbm, z1_hbm, ag_hbm, ai_hbm, dg_hbm, di_hbm,
              ib, rh, rt, ag, ai, dg_sh, di_sh, *sems):
    c, s, wid = _ids()
    st = pl.ds(s * STRIPE, STRIPE)
    pltpu.sync_copy(z1_hbm.at[st], dg_sh.at[st])
    pltpu.sync_copy(z1_hbm.at[st], di_sh.at[st])
    plsc.subcore_barrier()
    sgm, ssm = sems[:NBUF], sems[NBUF:]
    lane = lax.iota(i32, LANES)
    base = wid * NK

    def fi(k, b):
        pltpu.sync_copy(ht_hbm.at[base + k], ib.at[b])

    def ig(k, b):
        pltpu.async_copy(c2_hbm.at[ib.at[b, 0]], rh.at[b], sgm[b])
        pltpu.async_copy(c2_hbm.at[ib.at[b, 1]], rt.at[b], sgm[b])

    def wg(k, b):
        pltpu.make_async_copy(c2_hbm.at[ib.at[b, 0]], rh.at[b],
                              sgm[b]).wait()
        pltpu.make_async_copy(c2_hbm.at[ib.at[b, 1]], rt.at[b],
                              sgm[b]).wait()

    def cs(k, b):
        @pl.when(k >= 2)
        def _():
            kp = k - 2
            pltpu.make_async_copy(ag.at[b], ag_hbm.at[base + kp],
                                  ssm[b]).wait()
            pltpu.make_async_copy(ai.at[b], ai_hbm.at[base + kp],
                                  ssm[b]).wait()

        @pl.loop(0, CH // LANES)
        def _(g):
            agv = jnp.zeros((LANES,), f32)
            aiv = jnp.zeros((LANES,), f32)
            for j in range(LANES):
                e = g * LANES + j
                sgv = (rh[b, e, pl.ds(0, 16)] * rt[b, e, pl.ds(0, 16)]
                       + rh[b, e, pl.ds(16, 16)] * rt[b, e, pl.ds(16, 16)])
                siv = (rh[b, e, pl.ds(32, 16)] * rt[b, e, pl.ds(32, 16)]
                       + rh[b, e, pl.ds(48, 16)] * rt[b, e, pl.ds(48, 16)])
                agv = jnp.where(lane == j, (jnp.sum(sgv) + 1.0) * 0.5, agv)
                aiv = jnp.where(lane == j, (jnp.sum(siv) + 1.0) * 0.5, aiv)
            ag[b, pl.ds(g * LANES, LANES)] = agv
            ai[b, pl.ds(g * LANES, LANES)] = aiv

        pltpu.async_copy(ag.at[b], ag_hbm.at[base + k], ssm[b])
        pltpu.async_copy(ai.at[b], ai_hbm.at[base + k], ssm[b])
        pltpu.sync_copy(ag.at[b], dg_sh.at[ib.at[b, 0]], add=True)
        pltpu.sync_copy(ai.at[b], di_sh.at[ib.at[b, 0]], add=True)

    _ring(NK, fi, ig, wg, cs)
    for b in range(2):
        kp = NK - 2 + b
        pltpu.make_async_copy(ag.at[b], ag_hbm.at[base + kp], ssm[b]).wait()
        pltpu.make_async_copy(ai.at[b], ai_hbm.at[base + kp], ssm[b]).wait()
    plsc.subcore_barrier()
    pltpu.sync_copy(dg_sh.at[st], dg_hbm.at[c, st])
    pltpu.sync_copy(di_sh.at[st], di_hbm.at[c, st])


@functools.partial(
    pl.kernel,
    out_type=jax.ShapeDtypeStruct((NC, NPAD, D), f32),
    mesh=_MESH,
    compiler_params=_SC_PARAMS,
    scratch_types=[
        pltpu.VMEM((NBUF, 2, CH), i32),
        pltpu.VMEM((NBUF, CH, D), f32),
        pltpu.VMEM((NBUF, CH), f32),
        pltpu.VMEM((NBUF, CH), f32),
        pltpu.VMEM((NBUF, CH), f32),
        pltpu.VMEM((NBUF, CH), f32),
        pltpu.VMEM_SHARED((NPAD, D), f32),
    ] + _SEMS,
)
def _sc_gaia(ht_hbm, emb_hbm, dig_hbm, dii_hbm, ag_hbm, ai_hbm, z2_hbm,
             out_hbm, ib, rows, agb, aib, dgb, dib, acc_sh, *sems):
    c, s, wid = _ids()
    st = pl.ds(s * STRIPE, STRIPE)
    pltpu.sync_copy(z2_hbm.at[st], acc_sh.at[st])
    plsc.subcore_barrier()
    sgm, ssm = sems[:NBUF], sems[NBUF:]
    base = wid * NK

    def fi(k, b):
        pltpu.sync_copy(ht_hbm.at[base + k], ib.at[b])

    def ig(k, b):
        pltpu.async_copy(emb_hbm.at[ib.at[b, 1]], rows.at[b], sgm[b])
        pltpu.async_copy(dig_hbm.at[ib.at[b, 0]], dgb.at[b], sgm[b])
        pltpu.async_copy(dii_hbm.at[ib.at[b, 0]], dib.at[b], sgm[b])
        pltpu.async_copy(ag_hbm.at[base + k], agb.at[b], sgm[b])
        pltpu.async_copy(ai_hbm.at[base + k], aib.at[b], sgm[b])

    def wg(k, b):
        pltpu.make_async_copy(emb_hbm.at[ib.at[b, 1]], rows.at[b],
                              sgm[b]).wait()
        pltpu.make_async_copy(dig_hbm.at[ib.at[b, 0]], dgb.at[b],
                              sgm[b]).wait()
        pltpu.make_async_copy(dii_hbm.at[ib.at[b, 0]], dib.at[b],
                              sgm[b]).wait()
        pltpu.make_async_copy(ag_hbm.at[base + k], agb.at[b], sgm[b]).wait()
        pltpu.make_async_copy(ai_hbm.at[base + k], aib.at[b], sgm[b]).wait()

    def cs(k, b):
        @pl.loop(0, CH // LANES)
        def _(g):
            sl = pl.ds(g * LANES, LANES)
            vvec = (dgb[b, sl] * agb[b, sl] + dib[b, sl] * aib[b, sl])
            for j in range(LANES):
                e = g * LANES + j
                v = vvec[j]
                rows[b, e, pl.ds(0, 16)] = rows[b, e, pl.ds(0, 16)] * v
                rows[b, e, pl.ds(16, 16)] = rows[b, e, pl.ds(16, 16)] * v

        pltpu.sync_copy(rows.at[b], acc_sh.at[ib.at[b, 0]], add=True)

    _ring(NK, fi, ig, wg, cs)
    plsc.subcore_barrier()
    pltpu.sync_copy(acc_sh.at[st], out_hbm.at[c, st])


_B1 = 2000
_B2 = 1000


def _tc_pre_body(degp_ref, emb_ref, dis_ref, embs_ref):
    dp = degp_ref[...]
    deg = dp[0] + dp[1]
    safe = jnp.where(deg > 0, deg, 1.0)
    dis = jnp.where(deg > 0, lax.rsqrt(safe), 0.0)
    dis_ref[...] = dis
    embs_ref[...] = emb_ref[...] * dis


def _tc_pre(deg_parts, emb):
    return pl.pallas_call(
        _tc_pre_body,
        grid=(N // _B1,),
        in_specs=[
            pl.BlockSpec((NC, _B1, 1), lambda i: (0, i, 0)),
            pl.BlockSpec((_B1, D), lambda i: (i, 0)),
        ],
        out_specs=[
            pl.BlockSpec((_B1, 1), lambda i: (i, 0)),
            pl.BlockSpec((_B1, D), lambda i: (i, 0)),
        ],
        out_shape=[
            jax.ShapeDtypeStruct((N, 1), f32),
            jax.ShapeDtypeStruct((N, D), f32),
        ],
    )(deg_parts[:, :N].reshape(NC, N, 1), emb)


def _tc_int_body(emb_ref, wu_ref, wi_ref, int_ref, intn_ref):
    pid = pl.program_id(0)
    w = jnp.where(pid < N_USERS // _B2, wu_ref[...], wi_ref[...])
    x = emb_ref[...]
    logits = jnp.dot(x, w, preferred_element_type=f32)
    m = jnp.max(logits, axis=1, keepdims=True)
    p = jnp.exp(logits - m)
    sm = p / jnp.sum(p, axis=1, keepdims=True)
    it = lax.dot_general(sm, w, (((1,), (1,)), ((), ())),
                         preferred_element_type=f32)
    int_ref[...] = it
    n = jnp.sqrt(jnp.sum(it * it, axis=1, keepdims=True))
    intn_ref[...] = it / jnp.maximum(n, 1e-12)


def _tc_int(emb, wu, wi):
    return pl.pallas_call(
        _tc_int_body,
        grid=(N // _B2,),
        in_specs=[
            pl.BlockSpec((_B2, D), lambda i: (i, 0)),
            pl.BlockSpec((D, NI), lambda i: (0, 0)),
            pl.BlockSpec((D, NI), lambda i: (0, 0)),
        ],
        out_specs=[
            pl.BlockSpec((_B2, D), lambda i: (i, 0)),
            pl.BlockSpec((_B2, D), lambda i: (i, 0)),
        ],
        out_shape=[
            jax.ShapeDtypeStruct((N, D), f32),
            jax.ShapeDtypeStruct((N, D), f32),
        ],
    )(emb, wu, wi)


def _tc_mid_body(sg_ref, dis_ref, intn_ref, gnn_ref, c2_ref):
    sg = sg_ref[...]
    gnn = (sg[0] + sg[1]) * dis_ref[...]
    gnn_ref[...] = gnn
    n = jnp.sqrt(jnp.sum(gnn * gnn, axis=1, keepdims=True))
    gnnn = gnn / jnp.maximum(n, 1e-12)
    c2_ref[...] = jnp.concatenate([gnnn, intn_ref[...]], axis=1)


def _tc_mid(sg_parts, dis, intn):
    return pl.pallas_call(
        _tc_mid_body,
        grid=(N // _B1,),
        in_specs=[
            pl.BlockSpec((NC, _B1, D), lambda i: (0, i, 0)),
            pl.BlockSpec((_B1, 1), lambda i: (i, 0)),
            pl.BlockSpec((_B1, D), lambda i: (i, 0)),
        ],
        out_specs=[
            pl.BlockSpec((_B1, D), lambda i: (i, 0)),
            pl.BlockSpec((_B1, 2 * D), lambda i: (i, 0)),
        ],
        out_shape=[
            jax.ShapeDtypeStruct((N, D), f32),
            jax.ShapeDtypeStruct((N, 2 * D), f32),
        ],
    )(sg_parts, dis, intn)


def _tc_dinv_body(dgp_ref, dip_ref, dig_ref, dii_ref):
    dg = dgp_ref[...]
    di = dip_ref[...]
    g = dg[0] + dg[1]
    i = di[0] + di[1]
    dig_ref[...] = jnp.where(g != 0, 1.0 / jnp.where(g != 0, g, 1.0), 0.0)
    dii_ref[...] = jnp.where(i != 0, 1.0 / jnp.where(i != 0, i, 1.0), 0.0)


def _tc_dinv(dg_parts, di_parts):
    return pl.pallas_call(
        _tc_dinv_body,
        grid=(N // _B1,),
        in_specs=[
            pl.BlockSpec((NC, _B1, 1), lambda i: (0, i, 0)),
            pl.BlockSpec((NC, _B1, 1), lambda i: (0, i, 0)),
        ],
        out_specs=[
            pl.BlockSpec((_B1, 1), lambda i: (i, 0)),
            pl.BlockSpec((_B1, 1), lambda i: (i, 0)),
        ],
        out_shape=[
            jax.ShapeDtypeStruct((N, 1), f32),
            jax.ShapeDtypeStruct((N, 1), f32),
        ],
    )(dg_parts[:, :N].reshape(NC, N, 1),
      di_parts[:, :N].reshape(NC, N, 1))


def _tc_comb_body(gnn_ref, int_ref, ga_ref, emb_ref, acc_ref, dis_ref,
                  embn_ref, accn_ref, embsn_ref):
    ga = ga_ref[...]
    e2 = gnn_ref[...] + int_ref[...] + ga[0] + ga[1] + emb_ref[...]
    embn_ref[...] = e2
    accn_ref[...] = acc_ref[...] + e2
    embsn_ref[...] = e2 * dis_ref[...]


def _tc_comb(gnn, int_emb, ga_parts, emb, acc, dis):
    return pl.pallas_call(
        _tc_comb_body,
        grid=(N // _B1,),
        in_specs=[
            pl.BlockSpec((_B1, D), lambda i: (i, 0)),
            pl.BlockSpec((_B1, D), lambda i: (i, 0)),
            pl.BlockSpec((NC, _B1, D), lambda i: (0, i, 0)),
            pl.BlockSpec((_B1, D), lambda i: (i, 0)),
            pl.BlockSpec((_B1, D), lambda i: (i, 0)),
            pl.BlockSpec((_B1, 1), lambda i: (i, 0)),
        ],
        out_specs=[
            pl.BlockSpec((_B1, D), lambda i: (i, 0)),
            pl.BlockSpec((_B1, D), lambda i: (i, 0)),
            pl.BlockSpec((_B1, D), lambda i: (i, 0)),
        ],
        out_shape=[
            jax.ShapeDtypeStruct((N, D), f32),
            jax.ShapeDtypeStruct((N, D), f32),
            jax.ShapeDtypeStruct((N, D), f32),
        ],
    )(gnn, int_emb, ga_parts, emb, acc, dis)


def kernel(user_emb, item_emb, user_intent, item_intent, all_h_list,
           all_t_list):
    emb = jnp.concatenate([user_emb, item_emb], axis=0)
    h2 = jnp.pad(all_h_list, (0, EPAD - E),
                 constant_values=N).reshape(NCHP, CH)
    t2 = jnp.pad(all_t_list, (0, EPAD - E),
                 constant_values=0).reshape(NCHP, CH)
    ht2 = jnp.stack([h2, t2], axis=1)
    z1 = jnp.zeros((NPAD,), f32)
    z2 = jnp.zeros((NPAD, D), f32)

    deg_parts = _sc_deg(ht2, z1)
    dis, embs = _tc_pre(deg_parts, emb)

    acc = emb
    for _ in range(L):
        int_emb, intn = _tc_int(emb, user_intent, item_intent)
        sg_parts = _sc_spmm(ht2, embs, z2)
        gnn, c2 = _tc_mid(sg_parts[:, :N], dis, intn)
        ag, ai, dg_parts, di_parts = _sc_alpha(ht2, c2, z1)
        dig, dii = _tc_dinv(dg_parts, di_parts)
        dig_p = jnp.pad(dig.reshape(N), (0, NPAD - N))
        dii_p = jnp.pad(dii.reshape(N), (0, NPAD - N))
        ga_parts = _sc_gaia(ht2, emb, dig_p, dii_p, ag, ai, z2)
        emb, acc, embs = _tc_comb(gnn, int_emb, ga_parts[:, :N], emb,
                                   acc, dis)
    return acc
